# baseline (speedup 1.0000x reference)
import jax
import jax.numpy as jnp
from jax.experimental import pallas as pl
from jax.experimental.pallas import tpu as pltpu

_EPS = 1e-05


def _fused_kernel(x1_ref, x2_ref, w_ref, b_ref, gamma_ref, beta_ref,
                  out_add_ref, mean_ref, rstd_ref, out_ref):
    x = x1_ref[...] + x2_ref[...]
    out_add_ref[...] = x
    mean = jnp.mean(x, axis=1, keepdims=True)
    xc = x - mean
    var = jnp.mean(xc * xc, axis=1, keepdims=True)
    rstd = jax.lax.rsqrt(var + _EPS)
    mean_ref[...] = mean
    rstd_ref[...] = rstd
    ln = (xc * rstd) * gamma_ref[...] + beta_ref[...]
    out_ref[...] = (
        jnp.dot(ln.astype(jnp.bfloat16), w_ref[...],
                preferred_element_type=jnp.float32)
        + b_ref[...]
    )


def kernel(x1, x2, w, b, gamma, beta):
    B, M, N = x1.shape
    D = w.shape[1]
    R = B * M
    TM = 256

    x1f = x1.reshape(R, N)
    x2f = x2.reshape(R, N)
    wb = w.astype(jnp.bfloat16)
    b2 = b.reshape(1, D)
    gamma2 = gamma.reshape(1, N)
    beta2 = beta.reshape(1, N)

    out_add, mean, rstd, out = pl.pallas_call(
        _fused_kernel,
        grid=(R // TM,),
        in_specs=[
            pl.BlockSpec((TM, N), lambda i: (i, 0)),
            pl.BlockSpec((TM, N), lambda i: (i, 0)),
            pl.BlockSpec((N, D), lambda i: (0, 0)),
            pl.BlockSpec((1, D), lambda i: (0, 0)),
            pl.BlockSpec((1, N), lambda i: (0, 0)),
            pl.BlockSpec((1, N), lambda i: (0, 0)),
        ],
        out_specs=[
            pl.BlockSpec((TM, N), lambda i: (i, 0)),
            pl.BlockSpec((TM, 1), lambda i: (i, 0)),
            pl.BlockSpec((TM, 1), lambda i: (i, 0)),
            pl.BlockSpec((TM, D), lambda i: (i, 0)),
        ],
        out_shape=[
            jax.ShapeDtypeStruct((R, N), jnp.float32),
            jax.ShapeDtypeStruct((R, 1), jnp.float32),
            jax.ShapeDtypeStruct((R, 1), jnp.float32),
            jax.ShapeDtypeStruct((R, D), jnp.float32),
        ],
        compiler_params=pltpu.CompilerParams(
            dimension_semantics=("parallel",),
            vmem_limit_bytes=56 * 1024 * 1024,
        ),
        name="addln_matmul_fused",
    )(x1f, x2f, wb, b2, gamma2, beta2)

    return (
        out_add.reshape(B, M, N),
        mean.reshape(B, M),
        rstd.reshape(B, M),
        out.reshape(B, M, D),
    )


# speedup vs baseline: 1.0719x; 1.0719x over previous
import jax
import jax.numpy as jnp
from jax.experimental import pallas as pl
from jax.experimental.pallas import tpu as pltpu

_EPS = 1e-05


def _fused_kernel(x1_ref, x2_ref, w_ref, b_ref, gamma_ref, beta_ref,
                  out_add_ref, mean_ref, rstd_ref, out_ref):
    x = x1_ref[...] + x2_ref[...]
    out_add_ref[...] = x
    mean = jnp.mean(x, axis=1, keepdims=True)
    xc = x - mean
    var = jnp.mean(xc * xc, axis=1, keepdims=True)
    rstd = jax.lax.rsqrt(var + _EPS)
    mean_ref[...] = mean
    rstd_ref[...] = rstd
    ln = (xc * rstd) * gamma_ref[...] + beta_ref[...]
    out_ref[...] = (
        jnp.dot(ln, w_ref[...], preferred_element_type=jnp.float32)
        + b_ref[...]
    )


def kernel(x1, x2, w, b, gamma, beta):
    B, M, N = x1.shape
    D = w.shape[1]
    R = B * M
    TM = 256

    x1f = x1.reshape(R, N)
    x2f = x2.reshape(R, N)
    b2 = b.reshape(1, D)
    gamma2 = gamma.reshape(1, N)
    beta2 = beta.reshape(1, N)

    out_add, mean, rstd, out = pl.pallas_call(
        _fused_kernel,
        grid=(R // TM,),
        in_specs=[
            pl.BlockSpec((TM, N), lambda i: (i, 0)),
            pl.BlockSpec((TM, N), lambda i: (i, 0)),
            pl.BlockSpec((N, D), lambda i: (0, 0)),
            pl.BlockSpec((1, D), lambda i: (0, 0)),
            pl.BlockSpec((1, N), lambda i: (0, 0)),
            pl.BlockSpec((1, N), lambda i: (0, 0)),
        ],
        out_specs=[
            pl.BlockSpec((TM, N), lambda i: (i, 0)),
            pl.BlockSpec((TM, 1), lambda i: (i, 0)),
            pl.BlockSpec((TM, 1), lambda i: (i, 0)),
            pl.BlockSpec((TM, D), lambda i: (i, 0)),
        ],
        out_shape=[
            jax.ShapeDtypeStruct((R, N), jnp.float32),
            jax.ShapeDtypeStruct((R, 1), jnp.float32),
            jax.ShapeDtypeStruct((R, 1), jnp.float32),
            jax.ShapeDtypeStruct((R, D), jnp.float32),
        ],
        compiler_params=pltpu.CompilerParams(
            dimension_semantics=("parallel",),
            vmem_limit_bytes=56 * 1024 * 1024,
        ),
        name="addln_matmul_fused",
    )(x1f, x2f, w, b2, gamma2, beta2)

    return (
        out_add.reshape(B, M, N),
        mean.reshape(B, M),
        rstd.reshape(B, M),
        out.reshape(B, M, D),
    )


# speedup vs baseline: 1.1833x; 1.1039x over previous
import jax
import jax.numpy as jnp
from jax.experimental import pallas as pl
from jax.experimental.pallas import tpu as pltpu

_EPS = 1e-05


def _fused_kernel(x1_ref, x2_ref, w_ref, b_ref, gamma_ref, beta_ref,
                  out_add_ref, mean_ref, rstd_ref, out_ref):
    x = x1_ref[...] + x2_ref[...]
    out_add_ref[...] = x
    mean = jnp.mean(x, axis=1, keepdims=True)
    xc = x - mean
    var = jnp.mean(xc * xc, axis=1, keepdims=True)
    rstd = jax.lax.rsqrt(var + _EPS)
    mean_ref[...] = mean
    rstd_ref[...] = rstd
    ln = (xc * rstd) * gamma_ref[...] + beta_ref[...]
    out_ref[...] = (
        jnp.dot(ln, w_ref[...], preferred_element_type=jnp.float32)
        + b_ref[...]
    )


def kernel(x1, x2, w, b, gamma, beta):
    B, M, N = x1.shape
    D = w.shape[1]
    R = B * M
    TM = 512

    x1f = x1.reshape(R, N)
    x2f = x2.reshape(R, N)
    b2 = b.reshape(1, D)
    gamma2 = gamma.reshape(1, N)
    beta2 = beta.reshape(1, N)

    out_add, mean, rstd, out = pl.pallas_call(
        _fused_kernel,
        grid=(R // TM,),
        in_specs=[
            pl.BlockSpec((TM, N), lambda i: (i, 0)),
            pl.BlockSpec((TM, N), lambda i: (i, 0)),
            pl.BlockSpec((N, D), lambda i: (0, 0)),
            pl.BlockSpec((1, D), lambda i: (0, 0)),
            pl.BlockSpec((1, N), lambda i: (0, 0)),
            pl.BlockSpec((1, N), lambda i: (0, 0)),
        ],
        out_specs=[
            pl.BlockSpec((TM, N), lambda i: (i, 0)),
            pl.BlockSpec((TM, 1), lambda i: (i, 0)),
            pl.BlockSpec((TM, 1), lambda i: (i, 0)),
            pl.BlockSpec((TM, D), lambda i: (i, 0)),
        ],
        out_shape=[
            jax.ShapeDtypeStruct((R, N), jnp.float32),
            jax.ShapeDtypeStruct((R, 1), jnp.float32),
            jax.ShapeDtypeStruct((R, 1), jnp.float32),
            jax.ShapeDtypeStruct((R, D), jnp.float32),
        ],
        compiler_params=pltpu.CompilerParams(
            dimension_semantics=("parallel",),
            vmem_limit_bytes=56 * 1024 * 1024,
        ),
        name="addln_matmul_fused",
    )(x1f, x2f, w, b2, gamma2, beta2)

    return (
        out_add.reshape(B, M, N),
        mean.reshape(B, M),
        rstd.reshape(B, M),
        out.reshape(B, M, D),
    )


# speedup vs baseline: 1.2213x; 1.0322x over previous
import functools

import jax
import jax.numpy as jnp
from jax.experimental import pallas as pl
from jax.experimental.pallas import tpu as pltpu

_EPS = 1e-05


def _fused_kernel(x1_ref, x2_ref, w_ref, b_ref, gamma_ref, beta_ref,
                  out_add_ref, mean_ref, rstd_ref, out_ref, *, n_b):
    x = x1_ref[...] + x2_ref[...]
    out_add_ref[...] = x
    mean = jnp.mean(x, axis=1, keepdims=True)
    xc = x - mean
    var = jnp.mean(xc * xc, axis=1, keepdims=True)
    rstd = jax.lax.rsqrt(var + _EPS)
    brow = pl.program_id(0) % n_b
    mean_ref[pl.ds(brow, 1), :] = mean.reshape(1, -1)
    rstd_ref[pl.ds(brow, 1), :] = rstd.reshape(1, -1)
    ln = (xc * rstd) * gamma_ref[...] + beta_ref[...]
    out_ref[...] = (
        jnp.dot(ln, w_ref[...], preferred_element_type=jnp.float32)
        + b_ref[...]
    )


def kernel(x1, x2, w, b, gamma, beta):
    B, M, N = x1.shape
    D = w.shape[1]
    R = B * M
    TM = 512
    n_m = M // TM
    n_b = B

    x1f = x1.reshape(R, N)
    x2f = x2.reshape(R, N)
    b2 = b.reshape(1, D)
    gamma2 = gamma.reshape(1, N)
    beta2 = beta.reshape(1, N)

    def row_tile(i):
        return (i % n_b) * n_m + i // n_b

    body = functools.partial(_fused_kernel, n_b=n_b)

    out_add, mean, rstd, out = pl.pallas_call(
        body,
        grid=(R // TM,),
        in_specs=[
            pl.BlockSpec((TM, N), lambda i: (row_tile(i), 0)),
            pl.BlockSpec((TM, N), lambda i: (row_tile(i), 0)),
            pl.BlockSpec((N, D), lambda i: (0, 0)),
            pl.BlockSpec((1, D), lambda i: (0, 0)),
            pl.BlockSpec((1, N), lambda i: (0, 0)),
            pl.BlockSpec((1, N), lambda i: (0, 0)),
        ],
        out_specs=[
            pl.BlockSpec((TM, N), lambda i: (row_tile(i), 0)),
            pl.BlockSpec((B, TM), lambda i: (0, i // n_b)),
            pl.BlockSpec((B, TM), lambda i: (0, i // n_b)),
            pl.BlockSpec((TM, D), lambda i: (row_tile(i), 0)),
        ],
        out_shape=[
            jax.ShapeDtypeStruct((R, N), jnp.float32),
            jax.ShapeDtypeStruct((B, M), jnp.float32),
            jax.ShapeDtypeStruct((B, M), jnp.float32),
            jax.ShapeDtypeStruct((R, D), jnp.float32),
        ],
        compiler_params=pltpu.CompilerParams(
            dimension_semantics=("parallel",),
            vmem_limit_bytes=56 * 1024 * 1024,
        ),
        name="addln_matmul_fused",
    )(x1f, x2f, w, b2, gamma2, beta2)

    return (
        out_add.reshape(B, M, N),
        mean,
        rstd,
        out.reshape(B, M, D),
    )
